# baseline (speedup 1.0000x reference)
import functools
import jax
import jax.numpy as jnp
from jax import lax
from jax.experimental import pallas as pl
from jax.experimental.pallas import tpu as pltpu
from jax.experimental.pallas import tpu_sc as plsc

D_MODEL = 64
SCALE = 8.0
NUM_WORKERS = 32
CHUNK = 256
NBUF = 4
LANES = 16
ROWS_PER_ITER = 8


def _make_emb_kernel(B, V):
    b_per_w = B // NUM_WORKERS
    n_chunks = b_per_w // CHUNK
    assert n_chunks % NBUF == 0 and n_chunks >= 2 * NBUF
    mesh = plsc.VectorSubcoreMesh(core_axis_name="c", subcore_axis_name="s")

    scratch = (
        [pltpu.VMEM((b_per_w,), jnp.int32)]
        + [pltpu.VMEM((CHUNK, D_MODEL), jnp.float32) for _ in range(NBUF)]
        + [pltpu.SemaphoreType.DMA for _ in range(2 * NBUF)]
    )

    @functools.partial(
        pl.kernel,
        mesh=mesh,
        compiler_params=pltpu.CompilerParams(use_tc_tiling_on_sc=False),
        out_type=jax.ShapeDtypeStruct((B, D_MODEL), jnp.float32),
        scratch_types=scratch,
    )
    def emb_kernel(idx_hbm, table_hbm, out_hbm, idx_all, *bufs_and_sems):
        rows = bufs_and_sems[:NBUF]
        gsem = bufs_and_sems[NBUF : 2 * NBUF]
        osem = bufs_and_sems[2 * NBUF : 3 * NBUF]

        cid = lax.axis_index("c")
        sid = lax.axis_index("s")
        wid = sid * 2 + cid
        base = wid * b_per_w

        pltpu.sync_copy(idx_hbm.at[pl.ds(base, b_per_w)], idx_all)

        def gather_desc(c, b):
            return pltpu.make_async_copy(
                table_hbm.at[idx_all.at[pl.ds(c * CHUNK, CHUNK)]],
                rows[b],
                gsem[b],
            )

        def out_desc(c, b):
            return pltpu.make_async_copy(
                rows[b],
                out_hbm.at[pl.ds(base + c * CHUNK, CHUNK)],
                osem[b],
            )

        def scale(b):
            rv = rows[b]

            @pl.loop(0, CHUNK // ROWS_PER_ITER)
            def _(i):
                r0 = i * ROWS_PER_ITER
                for r in range(ROWS_PER_ITER):
                    for j in range(D_MODEL // LANES):
                        sl = pl.ds(j * LANES, LANES)
                        rv[r0 + r, sl] = rv[r0 + r, sl] * SCALE

        for b in range(NBUF):
            gather_desc(b, b).start()

        for b in range(NBUF):
            c = b
            gather_desc(c, b).wait()
            scale(b)
            out_desc(c, b).start()
            if c >= 1:
                pb = (c - 1) % NBUF
                out_desc(c - 1, pb).wait()
                gather_desc(c + NBUF - 1, pb).start()

        @pl.loop(NBUF, n_chunks - NBUF, step=NBUF)
        def _(g0):
            for b in range(NBUF):
                c = g0 + b
                gather_desc(c, b).wait()
                scale(b)
                out_desc(c, b).start()
                pb = (b - 1) % NBUF
                out_desc(c - 1, pb).wait()
                gather_desc(c + NBUF - 1, pb).start()

        g0 = n_chunks - NBUF
        for b in range(NBUF):
            c = g0 + b
            gather_desc(c, b).wait()
            scale(b)
            out_desc(c, b).start()
            if b == 0:
                pb = (b - 1) % NBUF
                out_desc(c - 1, pb).wait()
                gather_desc(c + NBUF - 1, pb).start()

        for b in range(NBUF):
            out_desc(n_chunks - NBUF + b, b).wait()

    return emb_kernel


def kernel(input, table):
    B0, S = input.shape
    B = B0 * S
    V = table.shape[0]
    idx = input.reshape(B).astype(jnp.int32)
    out = _make_emb_kernel(B, V)(idx, table)
    return out.reshape(B0, S, D_MODEL)


# speedup vs baseline: 1.3260x; 1.3260x over previous
import functools
import jax
import jax.numpy as jnp
from jax import lax
from jax.experimental import pallas as pl
from jax.experimental.pallas import tpu as pltpu
from jax.experimental.pallas import tpu_sc as plsc

D_MODEL = 64
SCALE = 8.0
NUM_WORKERS = 32
NBUF = 4
LANES = 16
ROWS_PER_ITER = 8


def _make_emb_kernel(B0, S, V):
    b_per_w = B0 // NUM_WORKERS
    chunk = S
    n_chunks = b_per_w
    flat_per_w = b_per_w * S
    assert n_chunks % NBUF == 0 and n_chunks >= 2 * NBUF
    mesh = plsc.VectorSubcoreMesh(core_axis_name="c", subcore_axis_name="s")

    scratch = (
        [pltpu.VMEM((flat_per_w,), jnp.int32)]
        + [pltpu.VMEM((chunk, D_MODEL), jnp.float32) for _ in range(NBUF)]
        + [pltpu.SemaphoreType.DMA for _ in range(2 * NBUF)]
    )

    @functools.partial(
        pl.kernel,
        mesh=mesh,
        compiler_params=pltpu.CompilerParams(use_tc_tiling_on_sc=False),
        out_type=jax.ShapeDtypeStruct((B0, S // 8, 8, 128), jnp.float32),
        scratch_types=scratch,
    )
    def emb_kernel(idx_hbm, table_hbm, out_hbm, idx_all, *bufs_and_sems):
        rows = bufs_and_sems[:NBUF]
        gsem = bufs_and_sems[NBUF : 2 * NBUF]
        osem = bufs_and_sems[2 * NBUF : 3 * NBUF]

        cid = lax.axis_index("c")
        sid = lax.axis_index("s")
        wid = sid * 2 + cid
        base_b = wid * b_per_w

        pltpu.sync_copy(idx_hbm.at[pl.ds(base_b * S, flat_per_w)], idx_all)

        def gather_desc(c, b):
            return pltpu.make_async_copy(
                table_hbm.at[idx_all.at[pl.ds(c * chunk, chunk)]],
                rows[b],
                gsem[b],
            )

        def out_descs(c, b):
            return [
                pltpu.make_async_copy(
                    rows[b].at[pl.ds(t * 8, 8)],
                    out_hbm.at[base_b + c, t, :, pl.ds(0, D_MODEL)],
                    osem[b],
                )
                for t in range(S // 8)
            ]

        def out_start(c, b):
            for d in out_descs(c, b):
                d.start()

        def out_wait(c, b):
            for d in out_descs(c, b):
                d.wait()

        def scale(b):
            rv = rows[b]

            @pl.loop(0, chunk // ROWS_PER_ITER)
            def _(i):
                r0 = i * ROWS_PER_ITER
                for r in range(ROWS_PER_ITER):
                    for j in range(D_MODEL // LANES):
                        sl = pl.ds(j * LANES, LANES)
                        rv[r0 + r, sl] = rv[r0 + r, sl] * SCALE

        for b in range(NBUF):
            gather_desc(b, b).start()

        for b in range(NBUF):
            c = b
            gather_desc(c, b).wait()
            scale(b)
            out_start(c, b)
            if c >= 1:
                pb = (c - 1) % NBUF
                out_wait(c - 1, pb)
                gather_desc(c + NBUF - 1, pb).start()

        @pl.loop(NBUF, n_chunks - NBUF, step=NBUF)
        def _(g0):
            for b in range(NBUF):
                c = g0 + b
                gather_desc(c, b).wait()
                scale(b)
                out_start(c, b)
                pb = (b - 1) % NBUF
                out_wait(c - 1, pb)
                gather_desc(c + NBUF - 1, pb).start()

        g0 = n_chunks - NBUF
        for b in range(NBUF):
            c = g0 + b
            gather_desc(c, b).wait()
            scale(b)
            out_start(c, b)
            if b == 0:
                pb = (b - 1) % NBUF
                out_wait(c - 1, pb)
                gather_desc(c + NBUF - 1, pb).start()

        for b in range(NBUF):
            out_wait(n_chunks - NBUF + b, b)

    return emb_kernel


def kernel(input, table):
    B0, S = input.shape
    V = table.shape[0]
    idx = input.reshape(B0 * S).astype(jnp.int32)
    out5 = _make_emb_kernel(B0, S, V)(idx, table)
    return out5.reshape(B0, S, 128)[:, :, :D_MODEL]


# speedup vs baseline: 1.4261x; 1.0755x over previous
import functools
import jax
import jax.numpy as jnp
from jax import lax
from jax.experimental import pallas as pl
from jax.experimental.pallas import tpu as pltpu
from jax.experimental.pallas import tpu_sc as plsc

D_MODEL = 64
SCALE = 8.0
NUM_WORKERS = 32
NBUF = 4
LANES = 16
ROWS_PER_ITER = 8


def _make_emb_kernel(B0, S, V):
    b_per_w = B0 // NUM_WORKERS
    chunk = S
    n_chunks = b_per_w
    flat_per_w = b_per_w * S
    assert n_chunks % NBUF == 0 and n_chunks >= 2 * NBUF
    mesh = plsc.VectorSubcoreMesh(core_axis_name="c", subcore_axis_name="s")

    scratch = (
        [pltpu.VMEM((flat_per_w,), jnp.int32)]
        + [pltpu.VMEM((chunk, D_MODEL), jnp.float32) for _ in range(NBUF)]
        + [pltpu.SemaphoreType.DMA for _ in range(2 * NBUF)]
    )

    @functools.partial(
        pl.kernel,
        mesh=mesh,
        compiler_params=pltpu.CompilerParams(use_tc_tiling_on_sc=False),
        out_type=jax.ShapeDtypeStruct((B0, S // 8, 8, 128), jnp.float32),
        scratch_types=scratch,
    )
    def emb_kernel(idx_hbm, table_hbm, out_hbm, idx_all, *bufs_and_sems):
        rows = bufs_and_sems[:NBUF]
        gsem = bufs_and_sems[NBUF : 2 * NBUF]
        osem = bufs_and_sems[2 * NBUF : 3 * NBUF]

        cid = lax.axis_index("c")
        sid = lax.axis_index("s")
        wid = sid * 2 + cid
        base_b = wid * b_per_w

        pltpu.sync_copy(idx_hbm.at[pl.ds(base_b * S, flat_per_w)], idx_all)

        def gather_desc(c, b):
            return pltpu.make_async_copy(
                table_hbm.at[idx_all.at[pl.ds(c * chunk, chunk)]],
                rows[b],
                gsem[b],
            )

        def out_descs(c, b):
            return [
                pltpu.make_async_copy(
                    rows[b].at[pl.ds(t * 8, 8)],
                    out_hbm.at[base_b + c, t, :, pl.ds(0, D_MODEL)],
                    osem[b],
                )
                for t in range(S // 8)
            ]

        def out_start(c, b):
            for d in out_descs(c, b):
                d.start()

        def out_wait(c, b):
            for d in out_descs(c, b):
                d.wait()

        def scale(b):
            rv = rows[b]

            @pl.loop(0, chunk // ROWS_PER_ITER)
            def _(i):
                r0 = i * ROWS_PER_ITER
                for r in range(ROWS_PER_ITER):
                    for j in range(D_MODEL // LANES):
                        sl = pl.ds(j * LANES, LANES)
                        rv[r0 + r, sl] = rv[r0 + r, sl] * SCALE

        for b in range(NBUF):
            gather_desc(b, b).start()

        for b in range(NBUF):
            c = b
            gather_desc(c, b).wait()
            scale(b)
            out_start(c, b)
            if c >= 1:
                pb = (c - 1) % NBUF
                out_wait(c - 1, pb)
                gather_desc(c + NBUF - 1, pb).start()

        @pl.loop(NBUF, n_chunks - NBUF, step=NBUF)
        def _(g0):
            for b in range(NBUF):
                c = g0 + b
                gather_desc(c, b).wait()
                scale(b)
                out_start(c, b)
                pb = (b - 1) % NBUF
                out_wait(c - 1, pb)
                gather_desc(c + NBUF - 1, pb).start()

        g0 = n_chunks - NBUF
        for b in range(NBUF):
            c = g0 + b
            gather_desc(c, b).wait()
            scale(b)
            out_start(c, b)
            if b == 0:
                pb = (b - 1) % NBUF
                out_wait(c - 1, pb)
                gather_desc(c + NBUF - 1, pb).start()

        for b in range(NBUF):
            out_wait(n_chunks - NBUF + b, b)

    return emb_kernel


def kernel(input, table):
    B0, S = input.shape
    V = table.shape[0]
    idx = input.reshape(B0 * S).astype(jnp.int32) * 2
    tab2 = jnp.pad(table, ((0, 0), (0, 128 - D_MODEL))).reshape(2 * V, D_MODEL)
    out5 = _make_emb_kernel(B0, S, 2 * V)(idx, tab2)
    return out5.reshape(B0, S, 128)[:, :, :D_MODEL]


# speedup vs baseline: 2.0205x; 1.4168x over previous
import functools
import jax
import jax.numpy as jnp
from jax import lax
from jax.experimental import pallas as pl
from jax.experimental.pallas import tpu as pltpu
from jax.experimental.pallas import tpu_sc as plsc

D_MODEL = 64
SCALE = 8.0
NUM_WORKERS = 32
NBUF = 4
LANES = 16
ROWS_PER_ITER = 8


def _make_emb_kernel(B0, S, V):
    b_per_w = B0 // NUM_WORKERS
    chunk = S
    n_chunks = b_per_w
    flat_per_w = b_per_w * S
    assert n_chunks % NBUF == 0 and n_chunks >= 2 * NBUF
    mesh = plsc.VectorSubcoreMesh(core_axis_name="c", subcore_axis_name="s")

    scratch = (
        [pltpu.VMEM((flat_per_w,), jnp.int32)]
        + [pltpu.VMEM((chunk, D_MODEL), jnp.float32) for _ in range(NBUF)]
        + [pltpu.SemaphoreType.DMA for _ in range(2 * NBUF)]
    )

    @functools.partial(
        pl.kernel,
        mesh=mesh,
        compiler_params=pltpu.CompilerParams(use_tc_tiling_on_sc=False),
        out_type=jax.ShapeDtypeStruct((B0, S // 8, 8, 128), jnp.float32),
        scratch_types=scratch,
    )
    def emb_kernel(idx_hbm, table_hbm, out_hbm, idx_all, *bufs_and_sems):
        rows = bufs_and_sems[:NBUF]
        gsem = bufs_and_sems[NBUF : 2 * NBUF]
        osem = bufs_and_sems[2 * NBUF : 3 * NBUF]

        cid = lax.axis_index("c")
        sid = lax.axis_index("s")
        wid = sid * 2 + cid
        base_b = wid * b_per_w

        pltpu.sync_copy(idx_hbm.at[pl.ds(base_b * S, flat_per_w)], idx_all)

        def gather_desc(c, b):
            return pltpu.make_async_copy(
                table_hbm.at[idx_all.at[pl.ds(c * chunk, chunk)]],
                rows[b],
                gsem[b],
            )

        def out_descs(c, b):
            return [
                pltpu.make_async_copy(
                    rows[b].at[pl.ds(t * 8, 8)],
                    out_hbm.at[base_b + c, t, :, pl.ds(0, D_MODEL)],
                    osem[b],
                )
                for t in range(S // 8)
            ]

        def out_start(c, b):
            for d in out_descs(c, b):
                d.start()

        def out_wait(c, b):
            for d in out_descs(c, b):
                d.wait()

        def scale(b):
            rv = rows[b]

            @pl.loop(0, chunk // ROWS_PER_ITER)
            def _(i):
                r0 = i * ROWS_PER_ITER
                for r in range(ROWS_PER_ITER):
                    for j in range(D_MODEL // LANES):
                        sl = pl.ds(j * LANES, LANES)
                        rv[r0 + r, sl] = rv[r0 + r, sl] * SCALE

        for b in range(NBUF):
            gather_desc(b, b).start()

        for b in range(NBUF):
            c = b
            gather_desc(c, b).wait()
            scale(b)
            out_start(c, b)
            if c >= 1:
                pb = (c - 1) % NBUF
                out_wait(c - 1, pb)
                gather_desc(c + NBUF - 1, pb).start()

        @pl.loop(NBUF, n_chunks - NBUF, step=NBUF)
        def _(g0):
            for b in range(NBUF):
                c = g0 + b
                gather_desc(c, b).wait()
                scale(b)
                out_start(c, b)
                pb = (b - 1) % NBUF
                out_wait(c - 1, pb)
                gather_desc(c + NBUF - 1, pb).start()

        g0 = n_chunks - NBUF
        for b in range(NBUF):
            c = g0 + b
            gather_desc(c, b).wait()
            scale(b)
            out_start(c, b)
            if b == 0:
                pb = (b - 1) % NBUF
                out_wait(c - 1, pb)
                gather_desc(c + NBUF - 1, pb).start()

        for b in range(NBUF):
            out_wait(n_chunks - NBUF + b, b)

    return emb_kernel


_PAD_ROWS = 8192


def _transpose_pad(tabT, V):
    def body(in_ref, out_ref):
        t = in_ref[...].T
        out_ref[:, :D_MODEL] = t
        out_ref[:, D_MODEL:] = jnp.zeros_like(t)

    return pl.pallas_call(
        body,
        grid=(pl.cdiv(V, _PAD_ROWS),),
        in_specs=[pl.BlockSpec((D_MODEL, _PAD_ROWS), lambda i: (0, i))],
        out_specs=pl.BlockSpec((_PAD_ROWS, 128), lambda i: (i, 0)),
        out_shape=jax.ShapeDtypeStruct((V, 128), jnp.float32),
    )(tabT)


def kernel(input, table):
    B0, S = input.shape
    V = table.shape[0]
    idx = input.reshape(B0 * S).astype(jnp.int32) * 2
    tab2 = _transpose_pad(table.T, V).reshape(2 * V, D_MODEL)
    out5 = _make_emb_kernel(B0, S, 2 * V)(idx, tab2)
    return out5.reshape(B0, S, 128)[:, :, :D_MODEL]


# speedup vs baseline: 2.0822x; 1.0306x over previous
import functools
import jax
import jax.numpy as jnp
from jax import lax
from jax.experimental import pallas as pl
from jax.experimental.pallas import tpu as pltpu
from jax.experimental.pallas import tpu_sc as plsc

D_MODEL = 64
SCALE = 8.0
NUM_WORKERS = 32
NBUF = 4
LANES = 16
ROWS_PER_ITER = 8


def _make_emb_kernel(B0, S, V):
    b_per_w = B0 // NUM_WORKERS
    chunk = S
    n_chunks = b_per_w
    flat_per_w = b_per_w * S
    assert n_chunks % NBUF == 0 and n_chunks >= 2 * NBUF
    mesh = plsc.VectorSubcoreMesh(core_axis_name="c", subcore_axis_name="s")

    scratch = (
        [pltpu.VMEM((flat_per_w,), jnp.int32)]
        + [pltpu.VMEM((chunk, D_MODEL), jnp.float32) for _ in range(NBUF)]
        + [pltpu.SemaphoreType.DMA for _ in range(2 * NBUF)]
    )

    @functools.partial(
        pl.kernel,
        mesh=mesh,
        compiler_params=pltpu.CompilerParams(use_tc_tiling_on_sc=False),
        out_type=jax.ShapeDtypeStruct((B0, S // 8, 8, 128), jnp.float32),
        scratch_types=scratch,
    )
    def emb_kernel(idx_hbm, table_hbm, out_hbm, idx_all, *bufs_and_sems):
        rows = bufs_and_sems[:NBUF]
        gsem = bufs_and_sems[NBUF : 2 * NBUF]
        osem = bufs_and_sems[2 * NBUF : 3 * NBUF]

        cid = lax.axis_index("c")
        sid = lax.axis_index("s")
        wid = sid * 2 + cid
        base_b = wid * b_per_w

        pltpu.sync_copy(idx_hbm.at[pl.ds(base_b * S, flat_per_w)], idx_all)

        def gather_desc(c, b):
            return pltpu.make_async_copy(
                table_hbm.at[idx_all.at[pl.ds(c * chunk, chunk)]],
                rows[b],
                gsem[b],
            )

        def out_descs(c, b):
            return [
                pltpu.make_async_copy(
                    rows[b].at[pl.ds(t * 8, 8)],
                    out_hbm.at[base_b + c, t, :, pl.ds(0, D_MODEL)],
                    osem[b],
                )
                for t in range(S // 8)
            ]

        def out_start(c, b):
            for d in out_descs(c, b):
                d.start()

        def out_wait(c, b):
            for d in out_descs(c, b):
                d.wait()

        def scale(b):
            rv = rows[b]

            @pl.loop(0, chunk // ROWS_PER_ITER)
            def _(i):
                r0 = i * ROWS_PER_ITER
                for r in range(ROWS_PER_ITER):
                    for j in range(D_MODEL // LANES):
                        sl = pl.ds(j * LANES, LANES)
                        rv[r0 + r, sl] = rv[r0 + r, sl] * SCALE

        for b in range(NBUF):
            gather_desc(b, b).start()

        for b in range(NBUF):
            c = b
            gather_desc(c, b).wait()
            scale(b)
            out_start(c, b)
            if c >= 1:
                pb = (c - 1) % NBUF
                out_wait(c - 1, pb)
                gather_desc(c + NBUF - 1, pb).start()

        @pl.loop(NBUF, n_chunks - NBUF, step=NBUF)
        def _(g0):
            for b in range(NBUF):
                c = g0 + b
                gather_desc(c, b).wait()
                scale(b)
                out_start(c, b)
                pb = (b - 1) % NBUF
                out_wait(c - 1, pb)
                gather_desc(c + NBUF - 1, pb).start()

        g0 = n_chunks - NBUF
        for b in range(NBUF):
            c = g0 + b
            gather_desc(c, b).wait()
            scale(b)
            out_start(c, b)
            if b == 0:
                pb = (b - 1) % NBUF
                out_wait(c - 1, pb)
                gather_desc(c + NBUF - 1, pb).start()

        for b in range(NBUF):
            out_wait(n_chunks - NBUF + b, b)

    return emb_kernel


_PAD_ROWS = 16384


def _transpose_pad(tabT, V):
    def body(in_ref, out_ref):
        t = in_ref[...].T
        out_ref[:, :D_MODEL] = t
        out_ref[:, D_MODEL:] = jnp.zeros_like(t)

    return pl.pallas_call(
        body,
        grid=(pl.cdiv(V, _PAD_ROWS),),
        in_specs=[pl.BlockSpec((D_MODEL, _PAD_ROWS), lambda i: (0, i))],
        out_specs=pl.BlockSpec((_PAD_ROWS, 128), lambda i: (i, 0)),
        out_shape=jax.ShapeDtypeStruct((V, 128), jnp.float32),
    )(tabT)


def kernel(input, table):
    B0, S = input.shape
    V = table.shape[0]
    idx = input.reshape(B0 * S).astype(jnp.int32) * 2
    tab2 = _transpose_pad(table.T, V).reshape(2 * V, D_MODEL)
    out5 = _make_emb_kernel(B0, S, 2 * V)(idx, tab2)
    return out5.reshape(B0, S, 128)[:, :, :D_MODEL]


# speedup vs baseline: 2.1078x; 1.0123x over previous
import functools
import jax
import jax.numpy as jnp
from jax import lax
from jax.experimental import pallas as pl
from jax.experimental.pallas import tpu as pltpu
from jax.experimental.pallas import tpu_sc as plsc

D_MODEL = 64
SCALE = 8.0
NUM_WORKERS = 32
NBUF = 4
LANES = 16
ROWS_PER_ITER = 8


def _make_emb_kernel(B0, S, V):
    b_per_w = B0 // NUM_WORKERS
    chunk = S
    n_chunks = b_per_w
    flat_per_w = b_per_w * S
    assert n_chunks % NBUF == 0 and n_chunks >= 2 * NBUF
    mesh = plsc.VectorSubcoreMesh(core_axis_name="c", subcore_axis_name="s")

    scratch = (
        [pltpu.VMEM((flat_per_w,), jnp.int32)]
        + [pltpu.VMEM((chunk, D_MODEL), jnp.float32) for _ in range(NBUF)]
        + [pltpu.SemaphoreType.DMA for _ in range(2 * NBUF)]
    )

    @functools.partial(
        pl.kernel,
        mesh=mesh,
        compiler_params=pltpu.CompilerParams(use_tc_tiling_on_sc=False),
        out_type=jax.ShapeDtypeStruct((B0, S // 8, 8, 128), jnp.float32),
        scratch_types=scratch,
    )
    def emb_kernel(idx_hbm, table_hbm, out_hbm, idx_all, *bufs_and_sems):
        rows = bufs_and_sems[:NBUF]
        gsem = bufs_and_sems[NBUF : 2 * NBUF]
        osem = bufs_and_sems[2 * NBUF : 3 * NBUF]

        cid = lax.axis_index("c")
        sid = lax.axis_index("s")
        wid = sid * 2 + cid
        base_b = wid * b_per_w

        pltpu.sync_copy(idx_hbm.at[pl.ds(base_b * S, flat_per_w)], idx_all)

        def gather_desc(c, b):
            return pltpu.make_async_copy(
                table_hbm.at[idx_all.at[pl.ds(c * chunk, chunk)]],
                rows[b],
                gsem[b],
            )

        def out_descs(c, b):
            return [
                pltpu.make_async_copy(
                    rows[b].at[pl.ds(t * 8, 8)],
                    out_hbm.at[base_b + c, t, :, pl.ds(0, D_MODEL)],
                    osem[b],
                )
                for t in range(S // 8)
            ]

        def out_start(c, b):
            for d in out_descs(c, b):
                d.start()

        def out_wait(c, b):
            for d in out_descs(c, b):
                d.wait()

        def scale(b):
            rv = rows[b]

            @pl.loop(0, chunk // ROWS_PER_ITER)
            def _(i):
                r0 = i * ROWS_PER_ITER
                for r in range(ROWS_PER_ITER):
                    for j in range(D_MODEL // LANES):
                        sl = pl.ds(j * LANES, LANES)
                        rv[r0 + r, sl] = rv[r0 + r, sl] * SCALE

        for b in range(NBUF):
            gather_desc(b, b).start()

        for b in range(NBUF):
            c = b
            gather_desc(c, b).wait()
            scale(b)
            out_start(c, b)
            if c >= 1:
                pb = (c - 1) % NBUF
                out_wait(c - 1, pb)
                gather_desc(c + NBUF - 1, pb).start()

        @pl.loop(NBUF, n_chunks - NBUF, step=NBUF)
        def _(g0):
            for b in range(NBUF):
                c = g0 + b
                gather_desc(c, b).wait()
                scale(b)
                out_start(c, b)
                pb = (b - 1) % NBUF
                out_wait(c - 1, pb)
                gather_desc(c + NBUF - 1, pb).start()

        g0 = n_chunks - NBUF
        for b in range(NBUF):
            c = g0 + b
            gather_desc(c, b).wait()
            scale(b)
            out_start(c, b)
            if b == 0:
                pb = (b - 1) % NBUF
                out_wait(c - 1, pb)
                gather_desc(c + NBUF - 1, pb).start()

        for b in range(NBUF):
            out_wait(n_chunks - NBUF + b, b)

    return emb_kernel


_PAD_ROWS = 32768


def _transpose_pad(tabT, V):
    def body(in_ref, out_ref):
        t = in_ref[...].T
        out_ref[:, :D_MODEL] = t
        out_ref[:, D_MODEL:] = jnp.zeros_like(t)

    return pl.pallas_call(
        body,
        grid=(pl.cdiv(V, _PAD_ROWS),),
        in_specs=[pl.BlockSpec((D_MODEL, _PAD_ROWS), lambda i: (0, i))],
        out_specs=pl.BlockSpec((_PAD_ROWS, 128), lambda i: (i, 0)),
        out_shape=jax.ShapeDtypeStruct((V, 128), jnp.float32),
    )(tabT)


def kernel(input, table):
    B0, S = input.shape
    V = table.shape[0]
    idx = input.reshape(B0 * S).astype(jnp.int32) * 2
    tab2 = _transpose_pad(table.T, V).reshape(2 * V, D_MODEL)
    out5 = _make_emb_kernel(B0, S, 2 * V)(idx, tab2)
    return out5.reshape(B0, S, 128)[:, :, :D_MODEL]
